# baseline (speedup 1.0000x reference)
import jax
import jax.numpy as jnp
from jax import lax
from jax.experimental import pallas as pl
from jax.experimental.pallas import tpu as pltpu
from jax.experimental.pallas import tpu_sc as plsc

B, H, W, C = 2, 96, 96, 64
OH, OW = 80, 88
NPIX = OH * OW
NTAP, NOFF = 25, 100
CPAD = 128
G = 2
N_ELEM = B * G * NTAP * NPIX
NC, NS = 2, 16
PPW = NPIX // 8
PVEC = PPW // 16


def _conv_body(vol_ref, w_ref, b_ref, off_ref):
    r = pl.program_id(1)
    acc = jnp.zeros((OW, CPAD), jnp.float32)
    for t in range(NTAP):
        i, j = t // 5, t % 5
        a = vol_ref[0, r + 4 * i, pl.ds(2 * j, OW), :]
        acc += jnp.dot(a, w_ref[t], preferred_element_type=jnp.float32)
    off_ref[0, 0] = acc + b_ref[0]


def _conv_call(vol, wp, bp):
    return pl.pallas_call(
        _conv_body,
        grid=(B, OH),
        in_specs=[
            pl.BlockSpec((1, H, W, C), lambda b, r: (b, 0, 0, 0)),
            pl.BlockSpec((NTAP, C, CPAD), lambda b, r: (0, 0, 0)),
            pl.BlockSpec((1, CPAD), lambda b, r: (0, 0)),
        ],
        out_specs=pl.BlockSpec((1, 1, OW, CPAD), lambda b, r: (b, r, 0, 0)),
        out_shape=jax.ShapeDtypeStruct((B, OH, OW, CPAD), jnp.float32),
    )(vol, wp, bp)


def _sum_body(vol_ref, s_ref):
    s = jnp.sum(vol_ref[0], axis=-1)
    s_ref[0] = jnp.concatenate([s, jnp.zeros((H, 32), jnp.float32)], axis=-1)


def _sum_call(vol):
    return pl.pallas_call(
        _sum_body,
        grid=(B,),
        in_specs=[pl.BlockSpec((1, H, W, C), lambda b: (b, 0, 0, 0))],
        out_specs=pl.BlockSpec((1, H, 128), lambda b: (b, 0, 0)),
        out_shape=jax.ShapeDtypeStruct((B, H, 128), jnp.float32),
    )(vol)


def _sc_body(off_hbm, s_hbm, out_hbm, off_a, off_b, s_v, out_v, sem, osem):
    cid = lax.axis_index("c")
    sid = lax.axis_index("s")
    wid = sid * NC + cid
    b = wid // 16
    g = (wid // 8) % 2
    w8 = wid % 8
    pix0 = b * NPIX + w8 * PPW
    QP = PPW // 5
    bufs = (off_a, off_b)
    cps = [pltpu.async_copy(off_hbm.at[pl.ds(pix0 + q * QP, QP)], bufs[q % 2], sem)
           for q in range(2)]
    pltpu.sync_copy(s_hbm.at[b], s_v)

    lane = lax.iota(jnp.int32, 16)
    kyA = ((lane // 5) * 4 - 8).astype(jnp.float32)
    kxA = ((lane % 5) * 2 - 4).astype(jnp.float32)
    kB = lane + 9
    kyB = ((kB // 5) * 4 - 8).astype(jnp.float32)
    kxB = ((kB % 5) * 2 - 4).astype(jnp.float32)
    OST = 880
    sA = lane * OST
    sB = kB * OST
    c0 = g * NTAP
    cA = c0 + lane
    cB = c0 + 9 + lane

    def make_step(off_h, q):
        def step(p):
            pabs = w8 * PPW + q * QP + p
            py = lax.div(jnp.full((16,), pabs, jnp.int32), OW)
            yf = (py + 8).astype(jnp.float32)
            pv = jnp.full((16,), p, jnp.int32)
            pl_ofs = q * QP + p
            for ky, kx, sk, cv in ((kyA, kxA, sA, cA), (kyB, kxB, sB, cB)):
                dyv = plsc.load_gather(off_h, [pv, cv])
                dxv = plsc.load_gather(off_h, [pv, cv + 50])
                rx = dyv + ky
                ry = dxv + kx
                x0 = rx.astype(jnp.int32)
                y0 = ry.astype(jnp.int32)
                x1 = x0 + 1
                y1 = y0 + 1
                y0c = jnp.clip(y0, 0, H - 1)
                y1c = jnp.clip(y1, 0, H - 1)
                x0c = jnp.clip(x0, 0, W - 1)
                x1c = jnp.clip(x1, 0, W - 1)
                p0 = plsc.load_gather(s_v, [y0c, x0c])
                p1 = plsc.load_gather(s_v, [y0c, x1c])
                p2 = plsc.load_gather(s_v, [y1c, x0c])
                p3 = plsc.load_gather(s_v, [y1c, x1c])
                y0f = y0c.astype(jnp.float32)
                y1f = y1c.astype(jnp.float32)
                x0f = x0c.astype(jnp.float32)
                x1f = x1c.astype(jnp.float32)
                w0 = (y1f - ry) * (x1f - rx)
                w1 = (y1f - yf) * (rx - x0f)
                w2 = (ry - y0f) * (x1f - rx)
                w3 = (ry - y0f) * (rx - x0f)
                val = p0 * w0 + p1 * w1 + p2 * w2 + p3 * w3
                plsc.store_scatter(out_v, [sk + pl_ofs], val)
        return step

    for q in range(5):
        cps[q].wait()
        plsc.parallel_loop(0, QP, unroll=8)(make_step(bufs[q % 2], q))
        if q + 2 < 5:
            cps.append(pltpu.async_copy(
                off_hbm.at[pl.ds(pix0 + (q + 2) * QP, QP)], bufs[q % 2], sem))

    copies = []
    for k in range(NTAP):
        plane = ((b * G + g) * NTAP + k) * NPIX + w8 * PPW
        copies.append(pltpu.async_copy(out_v.at[pl.ds(k * OST, PPW)],
                                       out_hbm.at[pl.ds(plane, PPW)], osem))
    for cp in copies:
        cp.wait()


def _sc_call(off2d, s):
    mesh = plsc.VectorSubcoreMesh(core_axis_name="c", subcore_axis_name="s",
                                  num_cores=NC, num_subcores=NS)
    f = pl.kernel(
        _sc_body,
        out_type=jax.ShapeDtypeStruct((N_ELEM,), jnp.float32),
        mesh=mesh,
        compiler_params=pltpu.CompilerParams(needs_layout_passes=False),
        scratch_types=[
            pltpu.VMEM((PPW // 5, 128), jnp.float32),
            pltpu.VMEM((PPW // 5, 128), jnp.float32),
            pltpu.VMEM((H, 128), jnp.float32),
            pltpu.VMEM((NTAP * 880,), jnp.float32),
            pltpu.SemaphoreType.DMA,
            pltpu.SemaphoreType.DMA,
        ],
    )
    return f(off2d, s)


@jax.jit
def kernel(volume, conv_kernel, conv_bias):
    wp = (conv_kernel.reshape(5, 5, C, NTAP, 2, G)
          .transpose(0, 1, 2, 4, 5, 3)
          .reshape(5, 5, C, NOFF)
          .reshape(NTAP, C, NOFF))
    wp = jnp.pad(wp, ((0, 0), (0, 0), (0, CPAD - NOFF)))
    bp = (conv_bias.reshape(NTAP, 2, G).transpose(1, 2, 0)
          .reshape(1, NOFF))
    bp = jnp.pad(bp, ((0, 0), (0, CPAD - NOFF)))
    off = _conv_call(volume, wp, bp)
    s = _sum_call(volume)
    out_flat = _sc_call(off.reshape(B * NPIX, CPAD), s)
    return (out_flat.reshape(B, G, 5, 5, OH, OW)
            .transpose(0, 1, 4, 5, 2, 3))


# speedup vs baseline: 1.1432x; 1.1432x over previous
import jax
import jax.numpy as jnp
from jax import lax
from jax.experimental import pallas as pl
from jax.experimental.pallas import tpu as pltpu
from jax.experimental.pallas import tpu_sc as plsc

B, H, W, C = 2, 96, 96, 64
OH, OW = 80, 88
NPIX = OH * OW
NTAP, NOFF = 25, 100
CPAD = 128
G = 2
N_ELEM = B * G * NTAP * NPIX
NC, NS = 2, 16
PPW = NPIX // 8
PVEC = PPW // 16


def _conv_body(vol_ref, w_ref, b_ref, off_ref, x_ref):
    r = pl.program_id(1)

    @pl.when(r == 0)
    def _():
        for j in range(5):
            x_ref[:, :, pl.ds(j * C, C)] = vol_ref[0, :, pl.ds(2 * j, OW), :]

    acc = jnp.zeros((OW, CPAD), jnp.float32)
    for i in range(5):
        a = x_ref[r + 4 * i]
        acc += jnp.dot(a, w_ref[i], preferred_element_type=jnp.float32)
    off_ref[0, 0] = acc + b_ref[0]


def _conv_call(vol, wp, bp):
    return pl.pallas_call(
        _conv_body,
        grid=(B, OH),
        in_specs=[
            pl.BlockSpec((1, H, W, C), lambda b, r: (b, 0, 0, 0)),
            pl.BlockSpec((5, 5 * C, CPAD), lambda b, r: (0, 0, 0)),
            pl.BlockSpec((1, CPAD), lambda b, r: (0, 0)),
        ],
        out_specs=pl.BlockSpec((1, 1, OW, CPAD), lambda b, r: (b, r, 0, 0)),
        out_shape=jax.ShapeDtypeStruct((B, OH, OW, CPAD), jnp.float32),
        scratch_shapes=[pltpu.VMEM((H, OW, 5 * C), jnp.float32)],
    )(vol, wp, bp)


def _sum_body(vol_ref, s_ref):
    s = jnp.sum(vol_ref[0], axis=-1)
    s_ref[0] = jnp.concatenate([s, jnp.zeros((H, 32), jnp.float32)], axis=-1)


def _sum_call(vol):
    return pl.pallas_call(
        _sum_body,
        grid=(B,),
        in_specs=[pl.BlockSpec((1, H, W, C), lambda b: (b, 0, 0, 0))],
        out_specs=pl.BlockSpec((1, H, 128), lambda b: (b, 0, 0)),
        out_shape=jax.ShapeDtypeStruct((B, H, 128), jnp.float32),
    )(vol)


def _sc_body(off_hbm, s_hbm, out_hbm, off_a, off_b, s_v, out_v, sem, osem):
    cid = lax.axis_index("c")
    sid = lax.axis_index("s")
    wid = sid * NC + cid
    b = wid // 16
    g = (wid // 8) % 2
    w8 = wid % 8
    pix0 = b * NPIX + w8 * PPW
    QP = PPW // 5
    bufs = (off_a, off_b)
    cps = [pltpu.async_copy(off_hbm.at[pl.ds(pix0 + q * QP, QP)], bufs[q % 2], sem)
           for q in range(2)]
    pltpu.sync_copy(s_hbm.at[b], s_v)

    lane = lax.iota(jnp.int32, 16)
    kyA = ((lane // 5) * 4 - 8).astype(jnp.float32)
    kxA = ((lane % 5) * 2 - 4).astype(jnp.float32)
    kB = lane + 9
    kyB = ((kB // 5) * 4 - 8).astype(jnp.float32)
    kxB = ((kB % 5) * 2 - 4).astype(jnp.float32)
    OST = 880
    sA = lane * OST
    sB = kB * OST
    c0 = g * NTAP
    cA = c0 + lane
    cB = c0 + 9 + lane

    def make_step(off_h, q):
        def step(p):
            pabs = w8 * PPW + q * QP + p
            py = lax.div(jnp.full((16,), pabs, jnp.int32), OW)
            yf = (py + 8).astype(jnp.float32)
            pv = jnp.full((16,), p, jnp.int32)
            pl_ofs = q * QP + p
            for ky, kx, sk, cv in ((kyA, kxA, sA, cA), (kyB, kxB, sB, cB)):
                dyv = plsc.load_gather(off_h, [pv, cv])
                dxv = plsc.load_gather(off_h, [pv, cv + 50])
                rx = dyv + ky
                ry = dxv + kx
                x0 = rx.astype(jnp.int32)
                y0 = ry.astype(jnp.int32)
                x1 = x0 + 1
                y1 = y0 + 1
                y0c = jnp.clip(y0, 0, H - 1)
                y1c = jnp.clip(y1, 0, H - 1)
                x0c = jnp.clip(x0, 0, W - 1)
                x1c = jnp.clip(x1, 0, W - 1)
                p0 = plsc.load_gather(s_v, [y0c, x0c])
                p1 = plsc.load_gather(s_v, [y0c, x1c])
                p2 = plsc.load_gather(s_v, [y1c, x0c])
                p3 = plsc.load_gather(s_v, [y1c, x1c])
                y0f = y0c.astype(jnp.float32)
                y1f = y1c.astype(jnp.float32)
                x0f = x0c.astype(jnp.float32)
                x1f = x1c.astype(jnp.float32)
                w0 = (y1f - ry) * (x1f - rx)
                w1 = (y1f - yf) * (rx - x0f)
                w2 = (ry - y0f) * (x1f - rx)
                w3 = (ry - y0f) * (rx - x0f)
                val = p0 * w0 + p1 * w1 + p2 * w2 + p3 * w3
                plsc.store_scatter(out_v, [sk + pl_ofs], val)
        return step

    for q in range(5):
        cps[q].wait()
        plsc.parallel_loop(0, QP, unroll=4)(make_step(bufs[q % 2], q))
        if q + 2 < 5:
            cps.append(pltpu.async_copy(
                off_hbm.at[pl.ds(pix0 + (q + 2) * QP, QP)], bufs[q % 2], sem))

    copies = []
    for k in range(NTAP):
        plane = ((b * G + g) * NTAP + k) * NPIX + w8 * PPW
        copies.append(pltpu.async_copy(out_v.at[pl.ds(k * OST, PPW)],
                                       out_hbm.at[pl.ds(plane, PPW)], osem))
    for cp in copies:
        cp.wait()


def _sc_call(off2d, s):
    mesh = plsc.VectorSubcoreMesh(core_axis_name="c", subcore_axis_name="s",
                                  num_cores=NC, num_subcores=NS)
    f = pl.kernel(
        _sc_body,
        out_type=jax.ShapeDtypeStruct((N_ELEM,), jnp.float32),
        mesh=mesh,
        compiler_params=pltpu.CompilerParams(needs_layout_passes=False),
        scratch_types=[
            pltpu.VMEM((PPW // 5, 128), jnp.float32),
            pltpu.VMEM((PPW // 5, 128), jnp.float32),
            pltpu.VMEM((H, 128), jnp.float32),
            pltpu.VMEM((NTAP * 880,), jnp.float32),
            pltpu.SemaphoreType.DMA,
            pltpu.SemaphoreType.DMA,
        ],
    )
    return f(off2d, s)


@jax.jit
def kernel(volume, conv_kernel, conv_bias):
    wp = (conv_kernel.reshape(5, 5, C, NTAP, 2, G)
          .transpose(0, 1, 2, 4, 5, 3)
          .reshape(5, 5, C, NOFF)
          .reshape(NTAP, C, NOFF))
    wp = jnp.pad(wp, ((0, 0), (0, 0), (0, CPAD - NOFF)))
    wp = wp.reshape(5, 5 * C, CPAD)
    bp = (conv_bias.reshape(NTAP, 2, G).transpose(1, 2, 0)
          .reshape(1, NOFF))
    bp = jnp.pad(bp, ((0, 0), (0, CPAD - NOFF)))
    off = _conv_call(volume, wp, bp)
    s = _sum_call(volume)
    out_flat = _sc_call(off.reshape(B * NPIX, CPAD), s)
    return (out_flat.reshape(B, G, 5, 5, OH, OW)
            .transpose(0, 1, 4, 5, 2, 3))


# speedup vs baseline: 1.2349x; 1.0801x over previous
import jax
import jax.numpy as jnp
from jax import lax
from jax.experimental import pallas as pl
from jax.experimental.pallas import tpu as pltpu
from jax.experimental.pallas import tpu_sc as plsc

B, H, W, C = 2, 96, 96, 64
OH, OW = 80, 88
NPIX = OH * OW
NTAP, NOFF = 25, 100
CPAD = 128
G = 2
N_ELEM = B * G * NTAP * NPIX
NC, NS = 2, 16
PPW = NPIX // 16


def _conv_body(vol_ref, w_ref, b_ref, off_ref, x_ref):
    r = pl.program_id(0)

    @pl.when(r == 0)
    def _():
        for j in range(5):
            x_ref[:, :, pl.ds(j * C, C)] = vol_ref[0, :, pl.ds(2 * j, OW), :]

    acc = jnp.zeros((OW, CPAD), jnp.float32)
    for i in range(5):
        a = x_ref[r + 4 * i]
        acc += jnp.dot(a, w_ref[i], preferred_element_type=jnp.float32)
    off_ref[0, 0] = acc + b_ref[0]


def _conv_call(vol1, wp, bp):
    return pl.pallas_call(
        _conv_body,
        grid=(OH,),
        in_specs=[
            pl.BlockSpec((1, H, W, C), lambda r: (0, 0, 0, 0)),
            pl.BlockSpec((5, 5 * C, CPAD), lambda r: (0, 0, 0)),
            pl.BlockSpec((1, CPAD), lambda r: (0, 0)),
        ],
        out_specs=pl.BlockSpec((1, 1, OW, CPAD), lambda r: (0, r, 0, 0)),
        out_shape=jax.ShapeDtypeStruct((1, OH, OW, CPAD), jnp.float32),
        scratch_shapes=[pltpu.VMEM((H, OW, 5 * C), jnp.float32)],
    )(vol1, wp, bp)


def _sum_body(vol_ref, s_ref):
    s = jnp.sum(vol_ref[0], axis=-1)
    s_ref[0] = jnp.concatenate([s, jnp.zeros((H, 32), jnp.float32)], axis=-1)


def _sum_call(vol):
    return pl.pallas_call(
        _sum_body,
        grid=(B,),
        in_specs=[pl.BlockSpec((1, H, W, C), lambda b: (b, 0, 0, 0))],
        out_specs=pl.BlockSpec((1, H, 128), lambda b: (b, 0, 0)),
        out_shape=jax.ShapeDtypeStruct((B, H, 128), jnp.float32),
    )(vol)


def _sc_body(off_hbm, s_hbm, out_hbm, off_a, off_b, s_v, out_v, sem, osem):
    cid = lax.axis_index("c")
    sid = lax.axis_index("s")
    wid = sid * NC + cid
    g = wid // 16
    w8 = wid % 16
    pix0 = w8 * PPW
    QP = PPW // 5
    bufs = (off_a, off_b)
    cps = [pltpu.async_copy(off_hbm.at[pl.ds(pix0 + q * QP, QP)], bufs[q % 2], sem)
           for q in range(2)]
    pltpu.sync_copy(s_hbm, s_v)

    lane = lax.iota(jnp.int32, 16)
    kyA = ((lane // 5) * 4 - 8).astype(jnp.float32)
    kxA = ((lane % 5) * 2 - 4).astype(jnp.float32)
    kB = lane + 9
    kyB = ((kB // 5) * 4 - 8).astype(jnp.float32)
    kxB = ((kB % 5) * 2 - 4).astype(jnp.float32)
    OST = 880
    sA = lane * OST
    sB = kB * OST
    c0 = g * NTAP
    cA = c0 + lane
    cB = c0 + 9 + lane

    def make_step(off_h, q):
        def step(p):
            pabs = w8 * PPW + q * QP + p
            py = lax.div(jnp.full((16,), pabs, jnp.int32), OW)
            yf = (py + 8).astype(jnp.float32)
            pv = jnp.full((16,), p, jnp.int32)
            pl_ofs = q * QP + p
            for ky, kx, sk, cv in ((kyA, kxA, sA, cA), (kyB, kxB, sB, cB)):
                dyv = plsc.load_gather(off_h, [pv, cv])
                dxv = plsc.load_gather(off_h, [pv, cv + 50])
                rx = dyv + ky
                ry = dxv + kx
                x0 = rx.astype(jnp.int32)
                y0 = ry.astype(jnp.int32)
                x1 = x0 + 1
                y1 = y0 + 1
                y0c = jnp.clip(y0, 0, H - 1)
                y1c = jnp.clip(y1, 0, H - 1)
                x0c = jnp.clip(x0, 0, W - 1)
                x1c = jnp.clip(x1, 0, W - 1)
                p0 = plsc.load_gather(s_v, [y0c, x0c])
                p1 = plsc.load_gather(s_v, [y0c, x1c])
                p2 = plsc.load_gather(s_v, [y1c, x0c])
                p3 = plsc.load_gather(s_v, [y1c, x1c])
                y0f = y0c.astype(jnp.float32)
                y1f = y1c.astype(jnp.float32)
                x0f = x0c.astype(jnp.float32)
                x1f = x1c.astype(jnp.float32)
                w0 = (y1f - ry) * (x1f - rx)
                w1 = (y1f - yf) * (rx - x0f)
                w2 = (ry - y0f) * (x1f - rx)
                w3 = (ry - y0f) * (rx - x0f)
                val = p0 * w0 + p1 * w1 + p2 * w2 + p3 * w3
                plsc.store_scatter(out_v, [sk + pl_ofs], val)
        return step

    for q in range(5):
        cps[q].wait()
        plsc.parallel_loop(0, QP, unroll=4)(make_step(bufs[q % 2], q))
        if q + 2 < 5:
            cps.append(pltpu.async_copy(
                off_hbm.at[pl.ds(pix0 + (q + 2) * QP, QP)], bufs[q % 2], sem))

    copies = []
    for k in range(NTAP):
        plane = (g * NTAP + k) * NPIX + w8 * PPW
        copies.append(pltpu.async_copy(out_v.at[pl.ds(k * OST, PPW)],
                                       out_hbm.at[pl.ds(plane, PPW)], osem))
    for cp in copies:
        cp.wait()


def _sc_call(off2d, s):
    mesh = plsc.VectorSubcoreMesh(core_axis_name="c", subcore_axis_name="s",
                                  num_cores=NC, num_subcores=NS)
    f = pl.kernel(
        _sc_body,
        out_type=jax.ShapeDtypeStruct((N_ELEM // B,), jnp.float32),
        mesh=mesh,
        compiler_params=pltpu.CompilerParams(needs_layout_passes=False),
        scratch_types=[
            pltpu.VMEM((PPW // 5, 128), jnp.float32),
            pltpu.VMEM((PPW // 5, 128), jnp.float32),
            pltpu.VMEM((H, 128), jnp.float32),
            pltpu.VMEM((NTAP * 880,), jnp.float32),
            pltpu.SemaphoreType.DMA,
            pltpu.SemaphoreType.DMA,
        ],
    )
    return f(off2d, s)


@jax.jit
def kernel(volume, conv_kernel, conv_bias):
    wp = (conv_kernel.reshape(5, 5, C, NTAP, 2, G)
          .transpose(0, 1, 2, 4, 5, 3)
          .reshape(5, 5, C, NOFF)
          .reshape(NTAP, C, NOFF))
    wp = jnp.pad(wp, ((0, 0), (0, 0), (0, CPAD - NOFF)))
    wp = wp.reshape(5, 5 * C, CPAD)
    bp = (conv_bias.reshape(NTAP, 2, G).transpose(1, 2, 0)
          .reshape(1, NOFF))
    bp = jnp.pad(bp, ((0, 0), (0, CPAD - NOFF)))
    s = _sum_call(volume)
    outs = []
    for b in range(B):
        off = _conv_call(volume[b:b + 1], wp, bp)
        outs.append(_sc_call(off.reshape(NPIX, CPAD), s[b]))
    out_flat = jnp.concatenate(outs)
    return (out_flat.reshape(B, G, 5, 5, OH, OW)
            .transpose(0, 1, 4, 5, 2, 3))


# speedup vs baseline: 1.5154x; 1.2272x over previous
import jax
import jax.numpy as jnp
from jax import lax
from jax.experimental import pallas as pl
from jax.experimental.pallas import tpu as pltpu
from jax.experimental.pallas import tpu_sc as plsc

B, H, W, C = 2, 96, 96, 64
OH, OW = 80, 88
NPIX = OH * OW
NTAP, NOFF = 25, 100
CPAD = 128
G = 2
N_ELEM = B * G * NTAP * NPIX
NC, NS = 2, 16
PPW = NPIX // 16


def _conv_body(vol_ref, w_ref, b_ref, off_ref, x_ref):
    r = pl.program_id(0)

    @pl.when(r == 0)
    def _():
        for j in range(5):
            x_ref[:, :, pl.ds(j * C, C)] = vol_ref[0, :, pl.ds(2 * j, OW), :]

    for rr in range(8):
        acc = jnp.zeros((OW, CPAD), jnp.float32)
        for i in range(5):
            a = x_ref[r * 8 + rr + 4 * i]
            acc += jnp.dot(a, w_ref[i], preferred_element_type=jnp.float32)
        off_ref[0, rr] = acc + b_ref[0]


def _conv_call(vol1, wp, bp):
    return pl.pallas_call(
        _conv_body,
        grid=(OH // 8,),
        in_specs=[
            pl.BlockSpec((1, H, W, C), lambda r: (0, 0, 0, 0)),
            pl.BlockSpec((5, 5 * C, CPAD), lambda r: (0, 0, 0)),
            pl.BlockSpec((1, CPAD), lambda r: (0, 0)),
        ],
        out_specs=pl.BlockSpec((1, 8, OW, CPAD), lambda r: (0, r, 0, 0)),
        out_shape=jax.ShapeDtypeStruct((1, OH, OW, CPAD), jnp.float32),
        scratch_shapes=[pltpu.VMEM((H, OW, 5 * C), jnp.float32)],
    )(vol1, wp, bp)


def _sum_body(vol_ref, s_ref):
    s = jnp.sum(vol_ref[0], axis=-1)
    s_ref[0] = jnp.concatenate([s, jnp.zeros((H, 32), jnp.float32)], axis=-1)


def _sum_call(vol):
    return pl.pallas_call(
        _sum_body,
        grid=(B,),
        in_specs=[pl.BlockSpec((1, H, W, C), lambda b: (b, 0, 0, 0))],
        out_specs=pl.BlockSpec((1, H, 128), lambda b: (b, 0, 0)),
        out_shape=jax.ShapeDtypeStruct((B, H, 128), jnp.float32),
    )(vol)


def _sc_body(off_hbm, s_hbm, out_hbm, off_a, off_b, s_v, out_v, sem, osem):
    cid = lax.axis_index("c")
    sid = lax.axis_index("s")
    wid = sid * NC + cid
    g = wid // 16
    w8 = wid % 16
    pix0 = w8 * PPW
    QP = PPW // 5
    bufs = (off_a, off_b)
    cps = [pltpu.async_copy(off_hbm.at[pl.ds(pix0 + q * QP, QP)], bufs[q % 2], sem)
           for q in range(2)]
    pltpu.sync_copy(s_hbm, s_v)

    lane = lax.iota(jnp.int32, 16)
    kyA = ((lane // 5) * 4 - 8).astype(jnp.float32)
    kxA = ((lane % 5) * 2 - 4).astype(jnp.float32)
    kB = lane + 9
    kyB = ((kB // 5) * 4 - 8).astype(jnp.float32)
    kxB = ((kB % 5) * 2 - 4).astype(jnp.float32)
    OST = 880
    sA = lane * OST
    sB = kB * OST
    c0 = g * NTAP
    cA = c0 + lane
    cB = c0 + 9 + lane

    def make_step(off_h, q):
        def step(p):
            pabs = w8 * PPW + q * QP + p
            py = lax.div(jnp.full((16,), pabs, jnp.int32), OW)
            yf = (py + 8).astype(jnp.float32)
            pv = jnp.full((16,), p, jnp.int32)
            pl_ofs = q * QP + p
            for ky, kx, sk, cv in ((kyA, kxA, sA, cA), (kyB, kxB, sB, cB)):
                dyv = plsc.load_gather(off_h, [pv, cv])
                dxv = plsc.load_gather(off_h, [pv, cv + 50])
                rx = dyv + ky
                ry = dxv + kx
                x0 = rx.astype(jnp.int32)
                y0 = ry.astype(jnp.int32)
                x1 = x0 + 1
                y1 = y0 + 1
                y0c = jnp.clip(y0, 0, H - 1)
                y1c = jnp.clip(y1, 0, H - 1)
                x0c = jnp.clip(x0, 0, W - 1)
                x1c = jnp.clip(x1, 0, W - 1)
                p0 = plsc.load_gather(s_v, [y0c, x0c])
                p1 = plsc.load_gather(s_v, [y0c, x1c])
                p2 = plsc.load_gather(s_v, [y1c, x0c])
                p3 = plsc.load_gather(s_v, [y1c, x1c])
                y0f = y0c.astype(jnp.float32)
                y1f = y1c.astype(jnp.float32)
                x0f = x0c.astype(jnp.float32)
                x1f = x1c.astype(jnp.float32)
                w0 = (y1f - ry) * (x1f - rx)
                w1 = (y1f - yf) * (rx - x0f)
                w2 = (ry - y0f) * (x1f - rx)
                w3 = (ry - y0f) * (rx - x0f)
                val = p0 * w0 + p1 * w1 + p2 * w2 + p3 * w3
                plsc.store_scatter(out_v, [sk + pl_ofs], val)
        return step

    for q in range(5):
        cps[q].wait()
        plsc.parallel_loop(0, QP, unroll=4)(make_step(bufs[q % 2], q))
        if q + 2 < 5:
            cps.append(pltpu.async_copy(
                off_hbm.at[pl.ds(pix0 + (q + 2) * QP, QP)], bufs[q % 2], sem))

    copies = []
    for k in range(NTAP):
        plane = (g * NTAP + k) * NPIX + w8 * PPW
        copies.append(pltpu.async_copy(out_v.at[pl.ds(k * OST, PPW)],
                                       out_hbm.at[pl.ds(plane, PPW)], osem))
    for cp in copies:
        cp.wait()


def _sc_call(off2d, s):
    mesh = plsc.VectorSubcoreMesh(core_axis_name="c", subcore_axis_name="s",
                                  num_cores=NC, num_subcores=NS)
    f = pl.kernel(
        _sc_body,
        out_type=jax.ShapeDtypeStruct((N_ELEM // B,), jnp.float32),
        mesh=mesh,
        compiler_params=pltpu.CompilerParams(needs_layout_passes=False),
        scratch_types=[
            pltpu.VMEM((PPW // 5, 128), jnp.float32),
            pltpu.VMEM((PPW // 5, 128), jnp.float32),
            pltpu.VMEM((H, 128), jnp.float32),
            pltpu.VMEM((NTAP * 880,), jnp.float32),
            pltpu.SemaphoreType.DMA,
            pltpu.SemaphoreType.DMA,
        ],
    )
    return f(off2d, s)


@jax.jit
def kernel(volume, conv_kernel, conv_bias):
    wp = (conv_kernel.reshape(5, 5, C, NTAP, 2, G)
          .transpose(0, 1, 2, 4, 5, 3)
          .reshape(5, 5, C, NOFF)
          .reshape(NTAP, C, NOFF))
    wp = jnp.pad(wp, ((0, 0), (0, 0), (0, CPAD - NOFF)))
    wp = wp.reshape(5, 5 * C, CPAD)
    bp = (conv_bias.reshape(NTAP, 2, G).transpose(1, 2, 0)
          .reshape(1, NOFF))
    bp = jnp.pad(bp, ((0, 0), (0, CPAD - NOFF)))
    s = _sum_call(volume)
    outs = []
    for b in range(B):
        off = _conv_call(volume[b:b + 1], wp, bp)
        outs.append(_sc_call(off.reshape(NPIX, CPAD), s[b]))
    out_flat = jnp.concatenate(outs)
    return (out_flat.reshape(B, G, 5, 5, OH, OW)
            .transpose(0, 1, 4, 5, 2, 3))


# speedup vs baseline: 1.5230x; 1.0050x over previous
import jax
import jax.numpy as jnp
from jax import lax
from jax.experimental import pallas as pl
from jax.experimental.pallas import tpu as pltpu
from jax.experimental.pallas import tpu_sc as plsc

B, H, W, C = 2, 96, 96, 64
OH, OW = 80, 88
NPIX = OH * OW
NTAP, NOFF = 25, 100
CPAD = 128
G = 2
N_ELEM = B * G * NTAP * NPIX
NC, NS = 2, 16
PPW = NPIX // 16


def _conv_body(vol_ref, w_ref, b_ref, off_ref, x_ref):
    r = pl.program_id(0)

    @pl.when(r == 0)
    def _():
        for j in range(5):
            x_ref[:, :, pl.ds(j * C, C)] = vol_ref[0, :, pl.ds(2 * j, OW), :]

    for rr in range(16):
        acc = jnp.zeros((OW, CPAD), jnp.float32)
        for i in range(5):
            a = x_ref[r * 16 + rr + 4 * i]
            acc += jnp.dot(a, w_ref[i], preferred_element_type=jnp.float32)
        off_ref[0, rr] = acc + b_ref[0]


def _conv_call(vol1, wp, bp):
    return pl.pallas_call(
        _conv_body,
        grid=(OH // 16,),
        in_specs=[
            pl.BlockSpec((1, H, W, C), lambda r: (0, 0, 0, 0)),
            pl.BlockSpec((5, 5 * C, CPAD), lambda r: (0, 0, 0)),
            pl.BlockSpec((1, CPAD), lambda r: (0, 0)),
        ],
        out_specs=pl.BlockSpec((1, 16, OW, CPAD), lambda r: (0, r, 0, 0)),
        out_shape=jax.ShapeDtypeStruct((1, OH, OW, CPAD), jnp.float32),
        scratch_shapes=[pltpu.VMEM((H, OW, 5 * C), jnp.float32)],
    )(vol1, wp, bp)


def _sum_body(vol_ref, s_ref):
    s = jnp.sum(vol_ref[0], axis=-1)
    s_ref[0] = jnp.concatenate([s, jnp.zeros((H, 32), jnp.float32)], axis=-1)


def _sum_call(vol):
    return pl.pallas_call(
        _sum_body,
        grid=(B,),
        in_specs=[pl.BlockSpec((1, H, W, C), lambda b: (b, 0, 0, 0))],
        out_specs=pl.BlockSpec((1, H, 128), lambda b: (b, 0, 0)),
        out_shape=jax.ShapeDtypeStruct((B, H, 128), jnp.float32),
    )(vol)


def _sc_body(off_hbm, s_hbm, out_hbm, off_a, off_b, s_v, out_v, sem, osem):
    cid = lax.axis_index("c")
    sid = lax.axis_index("s")
    wid = sid * NC + cid
    g = wid // 16
    w8 = wid % 16
    pix0 = w8 * PPW
    QP = PPW // 5
    bufs = (off_a, off_b)
    cps = [pltpu.async_copy(off_hbm.at[pl.ds(pix0 + q * QP, QP)], bufs[q % 2], sem)
           for q in range(2)]
    pltpu.sync_copy(s_hbm, s_v)

    lane = lax.iota(jnp.int32, 16)
    kyA = ((lane // 5) * 4 - 8).astype(jnp.float32)
    kxA = ((lane % 5) * 2 - 4).astype(jnp.float32)
    kB = lane + 9
    kyB = ((kB // 5) * 4 - 8).astype(jnp.float32)
    kxB = ((kB % 5) * 2 - 4).astype(jnp.float32)
    OST = 880
    sA = lane * OST
    sB = kB * OST
    c0 = g * NTAP
    cA = c0 + lane
    cB = c0 + 9 + lane

    def make_step(off_h, q):
        def step(p):
            pabs = w8 * PPW + q * QP + p
            py = lax.div(jnp.full((16,), pabs, jnp.int32), OW)
            yf = (py + 8).astype(jnp.float32)
            pv = jnp.full((16,), p, jnp.int32)
            pl_ofs = q * QP + p
            for ky, kx, sk, cv in ((kyA, kxA, sA, cA), (kyB, kxB, sB, cB)):
                dyv = plsc.load_gather(off_h, [pv, cv])
                dxv = plsc.load_gather(off_h, [pv, cv + 50])
                rx = dyv + ky
                ry = dxv + kx
                x0 = rx.astype(jnp.int32)
                y0 = ry.astype(jnp.int32)
                x1 = x0 + 1
                y1 = y0 + 1
                y0c = jnp.clip(y0, 0, H - 1)
                y1c = jnp.clip(y1, 0, H - 1)
                x0c = jnp.clip(x0, 0, W - 1)
                x1c = jnp.clip(x1, 0, W - 1)
                p0 = plsc.load_gather(s_v, [y0c, x0c])
                p1 = plsc.load_gather(s_v, [y0c, x1c])
                p2 = plsc.load_gather(s_v, [y1c, x0c])
                p3 = plsc.load_gather(s_v, [y1c, x1c])
                y0f = y0c.astype(jnp.float32)
                y1f = y1c.astype(jnp.float32)
                x0f = x0c.astype(jnp.float32)
                x1f = x1c.astype(jnp.float32)
                w0 = (y1f - ry) * (x1f - rx)
                w1 = (y1f - yf) * (rx - x0f)
                w2 = (ry - y0f) * (x1f - rx)
                w3 = (ry - y0f) * (rx - x0f)
                val = p0 * w0 + p1 * w1 + p2 * w2 + p3 * w3
                plsc.store_scatter(out_v, [sk + pl_ofs], val)
        return step

    for q in range(5):
        cps[q].wait()
        plsc.parallel_loop(0, QP, unroll=4)(make_step(bufs[q % 2], q))
        if q + 2 < 5:
            cps.append(pltpu.async_copy(
                off_hbm.at[pl.ds(pix0 + (q + 2) * QP, QP)], bufs[q % 2], sem))

    copies = []
    for k in range(NTAP):
        plane = (g * NTAP + k) * NPIX + w8 * PPW
        copies.append(pltpu.async_copy(out_v.at[pl.ds(k * OST, PPW)],
                                       out_hbm.at[pl.ds(plane, PPW)], osem))
    for cp in copies:
        cp.wait()


def _sc_call(off2d, s):
    mesh = plsc.VectorSubcoreMesh(core_axis_name="c", subcore_axis_name="s",
                                  num_cores=NC, num_subcores=NS)
    f = pl.kernel(
        _sc_body,
        out_type=jax.ShapeDtypeStruct((N_ELEM // B,), jnp.float32),
        mesh=mesh,
        compiler_params=pltpu.CompilerParams(needs_layout_passes=False),
        scratch_types=[
            pltpu.VMEM((PPW // 5, 128), jnp.float32),
            pltpu.VMEM((PPW // 5, 128), jnp.float32),
            pltpu.VMEM((H, 128), jnp.float32),
            pltpu.VMEM((NTAP * 880,), jnp.float32),
            pltpu.SemaphoreType.DMA,
            pltpu.SemaphoreType.DMA,
        ],
    )
    return f(off2d, s)


@jax.jit
def kernel(volume, conv_kernel, conv_bias):
    wp = (conv_kernel.reshape(5, 5, C, NTAP, 2, G)
          .transpose(0, 1, 2, 4, 5, 3)
          .reshape(5, 5, C, NOFF)
          .reshape(NTAP, C, NOFF))
    wp = jnp.pad(wp, ((0, 0), (0, 0), (0, CPAD - NOFF)))
    wp = wp.reshape(5, 5 * C, CPAD)
    bp = (conv_bias.reshape(NTAP, 2, G).transpose(1, 2, 0)
          .reshape(1, NOFF))
    bp = jnp.pad(bp, ((0, 0), (0, CPAD - NOFF)))
    s = _sum_call(volume)
    outs = []
    for b in range(B):
        off = _conv_call(volume[b:b + 1], wp, bp)
        outs.append(_sc_call(off.reshape(NPIX, CPAD), s[b]))
    out_flat = jnp.concatenate(outs)
    return (out_flat.reshape(B, G, 5, 5, OH, OW)
            .transpose(0, 1, 4, 5, 2, 3))
